# baseline (speedup 1.0000x reference)
import functools

import jax
import jax.numpy as jnp
from jax import lax
from jax.experimental import pallas as pl
from jax.experimental.pallas import tpu as pltpu
from jax.experimental.pallas import tpu_sc as plsc

_COMMIT = 0.25

_NC = 2
_NS = 16
_NW = _NC * _NS


def _vq_body(x_ref, e_ref, idx_ref, acc_ref):
    i = pl.program_id(0)
    x = x_ref[...]
    e = e_ref[...]
    K = e.shape[0]
    mm = lax.dot_general(
        x, e, (((1,), (1,)), ((), ())),
        preferred_element_type=jnp.float32)
    xn2 = jnp.sum(x * x, axis=1, keepdims=True)
    en2 = jnp.sum(e * e, axis=1)[None, :]
    dist = (xn2 + en2) - 2.0 * mm
    dmin = jnp.min(dist, axis=1, keepdims=True)
    iota = lax.broadcasted_iota(jnp.int32, dist.shape, 1)
    idx = jnp.min(jnp.where(dist == dmin, iota, jnp.int32(K)), axis=1)
    idx_ref[0, 0, :] = idx

    @pl.when(i == 0)
    def _():
        acc_ref[...] = jnp.zeros_like(acc_ref)

    acc_ref[...] += jnp.sum(dmin).reshape(1, 1)


def _make_sc_gather(N, K, D):
    b_per_w = N // _NW
    mesh = plsc.VectorSubcoreMesh(core_axis_name="c", subcore_axis_name="s")

    @functools.partial(
        pl.kernel, mesh=mesh,
        out_type=jax.ShapeDtypeStruct((N, 128), jnp.float32),
        scratch_types=[
            pltpu.VMEM((b_per_w,), jnp.int32),
            pltpu.VMEM((b_per_w, 128), jnp.float32),
            pltpu.SemaphoreType.DMA,
        ],
    )
    def sc_gather(table_hbm, idx_hbm, out_hbm, idx_v, rows_v, sem):
        wid = lax.axis_index("s") * _NC + lax.axis_index("c")
        base = wid * b_per_w
        pltpu.sync_copy(idx_hbm.at[pl.ds(base, b_per_w)], idx_v)
        pltpu.async_copy(table_hbm.at[idx_v], rows_v, sem).wait()
        pltpu.sync_copy(rows_v, out_hbm.at[pl.ds(base, b_per_w)])

    return sc_gather


def kernel(inputs, embeddings):
    B, L, D = inputs.shape
    K = embeddings.shape[0]
    flat = inputs.reshape(-1, D)
    N = flat.shape[0]
    Rb = 512
    NB = N // Rb

    idx3, acc = pl.pallas_call(
        _vq_body,
        grid=(NB,),
        in_specs=[
            pl.BlockSpec((Rb, D), lambda i: (i, 0)),
            pl.BlockSpec((K, D), lambda i: (0, 0)),
        ],
        out_specs=[
            pl.BlockSpec((1, 1, Rb), lambda i: (i, 0, 0)),
            pl.BlockSpec((1, 1), lambda i: (0, 0)),
        ],
        out_shape=[
            jax.ShapeDtypeStruct((NB, 1, Rb), jnp.int32),
            jax.ShapeDtypeStruct((1, 1), jnp.float32),
        ],
    )(flat, embeddings)

    idx_flat = idx3.reshape(N)
    table128 = jnp.pad(embeddings, ((0, 0), (0, 128 - D)))
    q = _make_sc_gather(N, K, D)(table128, idx_flat)

    quantized = q[:, :D].reshape(B, L, D)
    loss = acc[0, 0] * ((1.0 + _COMMIT) / (N * D))
    idx = idx_flat.reshape(B, L, 1)
    return (quantized, loss, idx)


# speedup vs baseline: 1.1793x; 1.1793x over previous
import jax
import jax.numpy as jnp
from jax import lax
from jax.experimental import pallas as pl

_COMMIT = 0.25


def _vq_body(x_ref, e_ref, q_ref, idx_ref, acc_ref):
    i = pl.program_id(0)
    x = x_ref[...]
    e = e_ref[...]
    mm = lax.dot_general(
        x, e, (((1,), (1,)), ((), ())),
        preferred_element_type=jnp.float32)
    xn2 = jnp.sum(x * x, axis=1, keepdims=True)
    en2 = jnp.sum(e * e, axis=1)[None, :]
    dist = (xn2 + en2) - 2.0 * mm
    idx = jnp.argmin(dist, axis=1).astype(jnp.int32)
    idx_ref[0, 0, :] = idx
    iota = lax.broadcasted_iota(jnp.int32, dist.shape, 1)
    onehot = (iota == idx[:, None]).astype(jnp.bfloat16)
    e_hi = e.astype(jnp.bfloat16)
    e_lo = (e - e_hi.astype(jnp.float32)).astype(jnp.bfloat16)
    dn = (((1,), (0,)), ((), ()))
    q = (lax.dot_general(onehot, e_hi, dn,
                         preferred_element_type=jnp.float32)
         + lax.dot_general(onehot, e_lo, dn,
                           preferred_element_type=jnp.float32))
    q_ref[...] = x + (q - x)

    @pl.when(i == 0)
    def _():
        acc_ref[...] = jnp.zeros_like(acc_ref)

    acc_ref[...] += jnp.sum((q - x) ** 2).reshape(1, 1)


def kernel(inputs, embeddings):
    B, L, D = inputs.shape
    K = embeddings.shape[0]
    flat = inputs.reshape(-1, D)
    N = flat.shape[0]
    Rb = 512
    NB = N // Rb

    q, idx3, acc = pl.pallas_call(
        _vq_body,
        grid=(NB,),
        in_specs=[
            pl.BlockSpec((Rb, D), lambda i: (i, 0)),
            pl.BlockSpec((K, D), lambda i: (0, 0)),
        ],
        out_specs=[
            pl.BlockSpec((Rb, D), lambda i: (i, 0)),
            pl.BlockSpec((1, 1, Rb), lambda i: (i, 0, 0)),
            pl.BlockSpec((1, 1), lambda i: (0, 0)),
        ],
        out_shape=[
            jax.ShapeDtypeStruct((N, D), jnp.float32),
            jax.ShapeDtypeStruct((NB, 1, Rb), jnp.int32),
            jax.ShapeDtypeStruct((1, 1), jnp.float32),
        ],
    )(flat, embeddings)

    quantized = q.reshape(B, L, D)
    loss = acc[0, 0] * ((1.0 + _COMMIT) / (N * D))
    idx = idx3.reshape(B, L, 1)
    return (quantized, loss, idx)
